# baseline (speedup 1.0000x reference)
import functools

import jax
import jax.numpy as jnp
from jax import lax
from jax.experimental import pallas as pl
from jax.experimental.pallas import tpu as pltpu
from jax.experimental.pallas import tpu_sc as plsc

K_WINNERS = 32
N_FEAT = 512
N_OUT = 20480
CHUNK = 128
N_CHUNK = N_OUT // CHUNK
CAPB = 32
NW = 32
ROWS_PER = 4096 // NW
CAND_MAX = N_CHUNK * CHUNK


def _iota16():
    return lax.iota(jnp.int32, 16)


def _select_rank(buf, sub1, sub2, hist2, hist, length, rank):
    iota = _iota16()
    lane_off = iota * 256
    acc_bits = jnp.int32(0)
    cur_len = length
    cur_rank = rank
    bufs = (buf, sub1, sub2, sub1)
    for lvl, (shift, width) in enumerate(
            ((23, 256), (15, 256), (7, 256), (0, 128))):
        src = bufs[lvl]
        mask_bits = width - 1
        ng = (cur_len + 15) // 16

        def hist_body(g, _, src=src, shift=shift, mask_bits=mask_bits,
                      cur_len=cur_len):
            kv = lax.bitcast_convert_type(src[pl.ds(g * 16, 16)], jnp.int32)
            b = (kv >> shift) & mask_bits
            valid = (g * 16 + iota) < cur_len
            plsc.addupdate_scatter(
                hist2, [lane_off + b], jnp.ones((16,), jnp.int32),
                mask=valid,
            )
            return 0

        lax.fori_loop(0, ng, hist_body, 0)

        nv = width // 16
        for v in range(nv):
            acc = hist2[pl.ds(v * 16, 16)]
            for l in range(1, 16):
                acc = acc + hist2[pl.ds(l * 256 + v * 16, 16)]
            hist[pl.ds(v * 16, 16)] = acc

        def zero_body(g, _, src=src, shift=shift, mask_bits=mask_bits):
            kv = lax.bitcast_convert_type(src[pl.ds(g * 16, 16)], jnp.int32)
            b = (kv >> shift) & mask_bits
            plsc.store_scatter(
                hist2, [lane_off + b], jnp.zeros((16,), jnp.int32)
            )
            return 0

        lax.fori_loop(0, ng, zero_body, 0)

        run = jnp.int32(0)
        bstar = jnp.int32(0)
        for v in range(nv - 1, -1, -1):
            h = hist[pl.ds(v * 16, 16)]
            sfx = lax.rev(plsc.cumsum(lax.rev(h, (0,))), (0,)) + run
            m = sfx >= cur_rank
            cand = jnp.where(m, iota + v * 16, -1)
            bstar = jnp.maximum(bstar, jnp.max(cand))
            run = run + jnp.sum(h)
        above = jnp.int32(0)
        for v in range(nv):
            h = hist[pl.ds(v * 16, 16)]
            above = above + jnp.sum(
                jnp.where((iota + v * 16) > bstar, h, 0)
            )
        cur_rank = cur_rank - above
        acc_bits = acc_bits | (bstar << shift)
        if lvl == 3:
            break
        dst = bufs[lvl + 1]

        def comp_body(g, off, src=src, dst=dst, shift=shift,
                      mask_bits=mask_bits, bstar=bstar, cur_len=cur_len):
            fv = src[pl.ds(g * 16, 16)]
            kv = lax.bitcast_convert_type(fv, jnp.int32)
            b = (kv >> shift) & mask_bits
            valid = (g * 16 + iota) < cur_len
            m = jnp.logical_and(b == bstar, valid)
            mi = m.astype(jnp.int32)
            pos = off + plsc.cumsum(mi) - 1
            plsc.store_scatter(dst, [pos], fv, mask=m)
            return off + jnp.sum(mi)

        cur_len = lax.fori_loop(0, ng, comp_body, jnp.int32(0))
    return acc_bits


def _sc_thresholds(M, yview, tlo):
    mesh = plsc.VectorSubcoreMesh(core_axis_name="c", subcore_axis_name="s")

    @functools.partial(
        pl.kernel,
        mesh=mesh,
        compiler_params=pltpu.CompilerParams(needs_layout_passes=False),
        out_type=jax.ShapeDtypeStruct((4096,), jnp.float32),
        scratch_types=[
            pltpu.VMEM((16, N_CHUNK), jnp.float32),
            pltpu.VMEM((16,), jnp.float32),
            pltpu.VMEM((N_CHUNK,), jnp.int32),
            pltpu.VMEM((CAPB, CHUNK), jnp.float32),
            pltpu.VMEM((CAND_MAX,), jnp.float32),
            pltpu.VMEM((CAND_MAX,), jnp.float32),
            pltpu.VMEM((CAND_MAX,), jnp.float32),
            pltpu.VMEM((4096,), jnp.int32),
            pltpu.VMEM((256,), jnp.int32),
            pltpu.VMEM((16,), jnp.float32),
            pltpu.SemaphoreType.DMA,
        ],
    )
    def sck(M_hbm, y_hbm, tlo_hbm, th_hbm,
            mbuf, tlobuf, idbuf, gbuf, candv, sub1, sub2, hist2, hist,
            tbuf, sem):
        iota = _iota16()
        wid = lax.axis_index("s") * 2 + lax.axis_index("c")
        base_row = wid * ROWS_PER
        for v in range(256):
            hist2[pl.ds(v * 16, 16)] = jnp.zeros((16,), jnp.int32)

        def group_body(g, _):
            gbase = base_row + g * 16
            pltpu.sync_copy(
                M_hbm.at[pl.ds(gbase, 16)], mbuf
            )
            pltpu.sync_copy(tlo_hbm.at[pl.ds(gbase, 16)], tlobuf)

            def row_body(i, tvec):
                row = gbase + i
                tlo = jnp.sum(jnp.where(iota == i, tlobuf[...], 0.0))
                for v in range(N_CHUNK // 16):
                    idbuf[pl.ds(v * 16, 16)] = jnp.zeros((16,), jnp.int32)

                def cid_body(v, off):
                    mv = mbuf[i, pl.ds(v * 16, 16)]
                    m = mv >= tlo
                    mi = m.astype(jnp.int32)
                    pos = off + plsc.cumsum(mi) - 1
                    gids = row * N_CHUNK + iota + v * 16
                    plsc.store_scatter(idbuf, [pos], gids, mask=m)
                    return off + jnp.sum(mi)

                ncand = lax.fori_loop(
                    0, N_CHUNK // 16, cid_body, jnp.int32(0)
                )
                nbatch = (ncand + CAPB - 1) // CAPB

                def batch_body(b, _):
                    pltpu.async_copy(
                        y_hbm.at[idbuf.at[pl.ds(b * CAPB, CAPB)]],
                        gbuf, sem,
                    ).wait()
                    base = b * CAPB * CHUNK
                    for s2 in range(CAPB):
                        for k in range(CHUNK // 16):
                            candv[pl.ds(base + s2 * CHUNK + k * 16, 16)] = (
                                gbuf[s2, pl.ds(k * 16, 16)]
                            )
                    return 0

                lax.fori_loop(0, nbatch, batch_body, 0)

                def pad_body(s, _):
                    for k in range(CHUNK // 16):
                        candv[pl.ds(s * CHUNK + k * 16, 16)] = (
                            jnp.zeros((16,), jnp.float32)
                        )
                    return 0

                lax.fori_loop(ncand, nbatch * CAPB, pad_body, 0)
                t_bits = _select_rank(
                    candv, sub1, sub2, hist2, hist, nbatch * CAPB * CHUNK,
                    jnp.int32(K_WINNERS),
                )
                tval = lax.bitcast_convert_type(t_bits, jnp.float32)
                return jnp.where(iota == i, tval, tvec)

            tvec = lax.fori_loop(
                0, 16, row_body, jnp.zeros((16,), jnp.float32)
            )
            tbuf[...] = tvec
            pltpu.sync_copy(tbuf, th_hbm.at[pl.ds(gbase, 16)])
            return 0

        lax.fori_loop(0, ROWS_PER // 16, group_body, 0)

    return sck(M, yview, tlo)


def _proj_body(x_ref, wp_ref, y_ref, m_ref, tlo_ref):
    y = lax.dot_general(
        x_ref[...], wp_ref[...],
        (((1,), (1,)), ((), ())),
        preferred_element_type=jnp.float32,
    )
    y_ref[...] = y
    m_ref[...] = jnp.max(
        y_ref[...].reshape(y.shape[0], N_CHUNK, CHUNK), axis=2
    )
    ysub = y_ref[:, :1280]
    hi0 = lax.bitcast_convert_type(
        jnp.max(ysub, axis=1, keepdims=True), jnp.int32
    ) + 1

    def body_a(_, carry):
        lo, hi = carry
        mid = lo + (hi - lo) // 2
        midf = lax.bitcast_convert_type(mid, jnp.float32)
        cnt = jnp.sum(
            (ysub >= midf).astype(jnp.float32), axis=1, keepdims=True
        )
        pred = cnt >= K_WINNERS
        return jnp.where(pred, mid, lo), jnp.where(pred, hi, mid)

    lo_a, _ = lax.fori_loop(0, 31, body_a, (jnp.zeros_like(hi0), hi0))
    tlo_ref[...] = lax.bitcast_convert_type(lo_a, jnp.float32)


def _final_body(y_ref, w2_ref, b2_ref, th_ref, out_ref):
    yv = y_ref[...]
    sparse = jnp.where(yv >= th_ref[...], yv, 0.0)
    w2c = jnp.maximum(w2_ref[...], 0.0)
    out = lax.dot_general(
        sparse, w2c,
        (((1,), (1,)), ((), ())),
        preferred_element_type=jnp.float32,
    )
    out_ref[...] = out + b2_ref[...]


@functools.partial(jax.jit, static_argnames=("bt",))
def _run(x, W_proj, W2, b2, bt=128):
    B, F = x.shape
    N = W_proj.shape[0]
    C = W2.shape[0]
    grid = (B // bt,)
    bt1 = 64
    y, M, tlo = pl.pallas_call(
        _proj_body,
        grid=(B // bt1,),
        in_specs=[
            pl.BlockSpec((bt1, F), lambda i: (i, 0)),
            pl.BlockSpec((N, F), lambda i: (0, 0)),
        ],
        out_specs=[
            pl.BlockSpec((bt1, N), lambda i: (i, 0)),
            pl.BlockSpec((bt1, N_CHUNK), lambda i: (i, 0)),
            pl.BlockSpec((bt1, 1), lambda i: (i, 0)),
        ],
        out_shape=[
            jax.ShapeDtypeStruct((B, N), jnp.float32),
            jax.ShapeDtypeStruct((B, N_CHUNK), jnp.float32),
            jax.ShapeDtypeStruct((B, 1), jnp.float32),
        ],
        compiler_params=pltpu.CompilerParams(
            vmem_limit_bytes=67000000,
        ),
    )(x, W_proj)
    thresh = _sc_thresholds(M, y.reshape(B * N_CHUNK, CHUNK), tlo.reshape(B))
    out = pl.pallas_call(
        _final_body,
        grid=grid,
        in_specs=[
            pl.BlockSpec((bt, N), lambda i: (i, 0)),
            pl.BlockSpec((C, N), lambda i: (0, 0)),
            pl.BlockSpec((1, C), lambda i: (0, 0)),
            pl.BlockSpec((bt, 1), lambda i: (i, 0)),
        ],
        out_specs=pl.BlockSpec((bt, C), lambda i: (i, 0)),
        out_shape=jax.ShapeDtypeStruct((B, C), jnp.float32),
        compiler_params=pltpu.CompilerParams(
            vmem_limit_bytes=67000000,
        ),
    )(y, W2, b2.reshape(1, C), thresh.reshape(B, 1))
    return out


def kernel(x, W_proj, W2, b2):
    return _run(x, W_proj, W2, b2)


# speedup vs baseline: 2.5908x; 2.5908x over previous
import functools

import jax
import jax.numpy as jnp
from jax import lax
from jax.experimental import pallas as pl
from jax.experimental.pallas import tpu as pltpu
from jax.experimental.pallas import tpu_sc as plsc

K_WINNERS = 32
N_FEAT = 512
N_OUT = 20480
CHUNK = 128
N_CHUNK = N_OUT // CHUNK
CAPB = 32
NW = 32
ROWS_PER = 4096 // NW
CAND_MAX = N_CHUNK * CHUNK


def _iota16():
    return lax.iota(jnp.int32, 16)


def _select_rank(buf, sub1, sub2, hist2, hist, length, rank):
    iota = _iota16()
    lane_off = iota * 256
    acc_bits = jnp.int32(0)
    cur_len = length
    cur_rank = rank
    bufs = (buf, sub1, sub2, sub1)
    for lvl, (shift, width) in enumerate(
            ((23, 256), (15, 256), (7, 256), (0, 128))):
        src = bufs[lvl]
        mask_bits = width - 1
        ng = (cur_len + 15) // 16

        def hist_body(g, _, src=src, shift=shift, mask_bits=mask_bits,
                      cur_len=cur_len):
            kv = lax.bitcast_convert_type(src[pl.ds(g * 16, 16)], jnp.int32)
            b = (kv >> shift) & mask_bits
            valid = (g * 16 + iota) < cur_len
            plsc.addupdate_scatter(
                hist2, [lane_off + b], jnp.ones((16,), jnp.int32),
                mask=valid,
            )
            return 0

        lax.fori_loop(0, ng, hist_body, 0)

        nv = width // 16
        for v in range(nv):
            acc = hist2[pl.ds(v * 16, 16)]
            for l in range(1, 16):
                acc = acc + hist2[pl.ds(l * 256 + v * 16, 16)]
            hist[pl.ds(v * 16, 16)] = acc

        def zero_body(g, _, src=src, shift=shift, mask_bits=mask_bits):
            kv = lax.bitcast_convert_type(src[pl.ds(g * 16, 16)], jnp.int32)
            b = (kv >> shift) & mask_bits
            plsc.store_scatter(
                hist2, [lane_off + b], jnp.zeros((16,), jnp.int32)
            )
            return 0

        lax.fori_loop(0, ng, zero_body, 0)

        run = jnp.int32(0)
        bstar = jnp.int32(0)
        for v in range(nv - 1, -1, -1):
            h = hist[pl.ds(v * 16, 16)]
            sfx = lax.rev(plsc.cumsum(lax.rev(h, (0,))), (0,)) + run
            m = sfx >= cur_rank
            cand = jnp.where(m, iota + v * 16, -1)
            bstar = jnp.maximum(bstar, jnp.max(cand))
            run = run + jnp.sum(h)
        above = jnp.int32(0)
        for v in range(nv):
            h = hist[pl.ds(v * 16, 16)]
            above = above + jnp.sum(
                jnp.where((iota + v * 16) > bstar, h, 0)
            )
        cur_rank = cur_rank - above
        acc_bits = acc_bits | (bstar << shift)
        if lvl == 3:
            break
        dst = bufs[lvl + 1]

        def comp_body(g, off, src=src, dst=dst, shift=shift,
                      mask_bits=mask_bits, bstar=bstar, cur_len=cur_len):
            fv = src[pl.ds(g * 16, 16)]
            kv = lax.bitcast_convert_type(fv, jnp.int32)
            b = (kv >> shift) & mask_bits
            valid = (g * 16 + iota) < cur_len
            m = jnp.logical_and(b == bstar, valid)
            mi = m.astype(jnp.int32)
            pos = off + plsc.cumsum(mi) - 1
            plsc.store_scatter(dst, [pos], fv, mask=m)
            return off + jnp.sum(mi)

        cur_len = lax.fori_loop(0, ng, comp_body, jnp.int32(0))
    return acc_bits


def _sc_thresholds(M, yview, tlo):
    mesh = plsc.VectorSubcoreMesh(core_axis_name="c", subcore_axis_name="s")

    @functools.partial(
        pl.kernel,
        mesh=mesh,
        compiler_params=pltpu.CompilerParams(needs_layout_passes=False),
        out_type=jax.ShapeDtypeStruct((4096,), jnp.float32),
        scratch_types=[
            pltpu.VMEM((16, N_CHUNK), jnp.float32),
            pltpu.VMEM((16,), jnp.float32),
            pltpu.VMEM((N_CHUNK,), jnp.int32),
            pltpu.VMEM((CAPB, CHUNK), jnp.float32),
            pltpu.VMEM((CAND_MAX,), jnp.float32),
            pltpu.VMEM((CAND_MAX,), jnp.float32),
            pltpu.VMEM((CAND_MAX,), jnp.float32),
            pltpu.VMEM((4096,), jnp.int32),
            pltpu.VMEM((256,), jnp.int32),
            pltpu.VMEM((16,), jnp.float32),
            pltpu.SemaphoreType.DMA,
        ],
    )
    def sck(M_hbm, y_hbm, tlo_hbm, th_hbm,
            mbuf, tlobuf, idbuf, gbuf, candv, sub1, sub2, hist2, hist,
            tbuf, sem):
        iota = _iota16()
        wid = lax.axis_index("s") * 2 + lax.axis_index("c")
        base_row = wid * ROWS_PER
        for v in range(256):
            hist2[pl.ds(v * 16, 16)] = jnp.zeros((16,), jnp.int32)

        def group_body(g, _):
            gbase = base_row + g * 16
            pltpu.sync_copy(
                M_hbm.at[pl.ds(gbase, 16)], mbuf
            )
            pltpu.sync_copy(tlo_hbm.at[pl.ds(gbase, 16)], tlobuf)

            def row_body(i, tvec):
                row = gbase + i
                tlo = jnp.sum(jnp.where(iota == i, tlobuf[...], 0.0))
                for v in range(N_CHUNK // 16):
                    idbuf[pl.ds(v * 16, 16)] = jnp.zeros((16,), jnp.int32)

                def cid_body(v, off):
                    mv = mbuf[i, pl.ds(v * 16, 16)]
                    m = mv >= tlo
                    mi = m.astype(jnp.int32)
                    pos = off + plsc.cumsum(mi) - 1
                    gids = row * N_CHUNK + iota + v * 16
                    plsc.store_scatter(idbuf, [pos], gids, mask=m)
                    return off + jnp.sum(mi)

                ncand = lax.fori_loop(
                    0, N_CHUNK // 16, cid_body, jnp.int32(0)
                )
                nbatch = (ncand + CAPB - 1) // CAPB

                def batch_body(b, _):
                    pltpu.async_copy(
                        y_hbm.at[idbuf.at[pl.ds(b * CAPB, CAPB)]],
                        gbuf, sem,
                    ).wait()
                    base = b * CAPB * CHUNK
                    for s2 in range(CAPB):
                        for k in range(CHUNK // 16):
                            candv[pl.ds(base + s2 * CHUNK + k * 16, 16)] = (
                                gbuf[s2, pl.ds(k * 16, 16)]
                            )
                    return 0

                lax.fori_loop(0, nbatch, batch_body, 0)

                def pad_body(s, _):
                    for k in range(CHUNK // 16):
                        candv[pl.ds(s * CHUNK + k * 16, 16)] = (
                            jnp.zeros((16,), jnp.float32)
                        )
                    return 0

                lax.fori_loop(ncand, nbatch * CAPB, pad_body, 0)
                t_bits = _select_rank(
                    candv, sub1, sub2, hist2, hist, nbatch * CAPB * CHUNK,
                    jnp.int32(K_WINNERS),
                )
                tval = lax.bitcast_convert_type(t_bits, jnp.float32)
                return jnp.where(iota == i, tval, tvec)

            tvec = lax.fori_loop(
                0, 16, row_body, jnp.zeros((16,), jnp.float32)
            )
            tbuf[...] = tvec
            pltpu.sync_copy(tbuf, th_hbm.at[pl.ds(gbase, 16)])
            return 0

        lax.fori_loop(0, ROWS_PER // 16, group_body, 0)

    return sck(M, yview, tlo)


def _proj_body(x_ref, wp_ref, y_ref, m_ref, tlo_ref):
    y = lax.dot_general(
        x_ref[...], wp_ref[...],
        (((1,), (1,)), ((), ())),
        preferred_element_type=jnp.float32,
    )
    y_ref[...] = y
    m_ref[...] = jnp.max(
        y_ref[...].reshape(y.shape[0], N_CHUNK, CHUNK), axis=2
    )
    ysub = m_ref[...]
    hi0 = lax.bitcast_convert_type(
        jnp.max(ysub, axis=1, keepdims=True), jnp.int32
    ) + 1

    def body_a(_, carry):
        lo, hi = carry
        mid = lo + (hi - lo) // 2
        midf = lax.bitcast_convert_type(mid, jnp.float32)
        cnt = jnp.sum(
            (ysub >= midf).astype(jnp.float32), axis=1, keepdims=True
        )
        pred = cnt >= K_WINNERS
        return jnp.where(pred, mid, lo), jnp.where(pred, hi, mid)

    lo_a, _ = lax.fori_loop(0, 31, body_a, (jnp.zeros_like(hi0), hi0))
    tlo_ref[...] = lax.bitcast_convert_type(lo_a, jnp.float32)


def _final_body(y_ref, w2_ref, b2_ref, th_ref, out_ref):
    yv = y_ref[...]
    sparse = jnp.where(yv >= th_ref[...], yv, 0.0)
    w2c = jnp.maximum(w2_ref[...], 0.0)
    out = lax.dot_general(
        sparse, w2c,
        (((1,), (1,)), ((), ())),
        preferred_element_type=jnp.float32,
    )
    out_ref[...] = out + b2_ref[...]


@functools.partial(jax.jit, static_argnames=("bt",))
def _run(x, W_proj, W2, b2, bt=128):
    B, F = x.shape
    N = W_proj.shape[0]
    C = W2.shape[0]
    grid = (B // bt,)
    bt1 = 64
    y, M, tlo = pl.pallas_call(
        _proj_body,
        grid=(B // bt1,),
        in_specs=[
            pl.BlockSpec((bt1, F), lambda i: (i, 0)),
            pl.BlockSpec((N, F), lambda i: (0, 0)),
        ],
        out_specs=[
            pl.BlockSpec((bt1, N), lambda i: (i, 0)),
            pl.BlockSpec((bt1, N_CHUNK), lambda i: (i, 0)),
            pl.BlockSpec((bt1, 1), lambda i: (i, 0)),
        ],
        out_shape=[
            jax.ShapeDtypeStruct((B, N), jnp.float32),
            jax.ShapeDtypeStruct((B, N_CHUNK), jnp.float32),
            jax.ShapeDtypeStruct((B, 1), jnp.float32),
        ],
        compiler_params=pltpu.CompilerParams(
            vmem_limit_bytes=67000000,
        ),
    )(x, W_proj)
    thresh = _sc_thresholds(M, y.reshape(B * N_CHUNK, CHUNK), tlo.reshape(B))
    out = pl.pallas_call(
        _final_body,
        grid=grid,
        in_specs=[
            pl.BlockSpec((bt, N), lambda i: (i, 0)),
            pl.BlockSpec((C, N), lambda i: (0, 0)),
            pl.BlockSpec((1, C), lambda i: (0, 0)),
            pl.BlockSpec((bt, 1), lambda i: (i, 0)),
        ],
        out_specs=pl.BlockSpec((bt, C), lambda i: (i, 0)),
        out_shape=jax.ShapeDtypeStruct((B, C), jnp.float32),
        compiler_params=pltpu.CompilerParams(
            vmem_limit_bytes=67000000,
        ),
    )(y, W2, b2.reshape(1, C), thresh.reshape(B, 1))
    return out


def kernel(x, W_proj, W2, b2):
    return _run(x, W_proj, W2, b2)
